# Initial kernel scaffold; baseline (speedup 1.0000x reference)
import functools

import jax
import jax.numpy as jnp
from jax import lax
from jax.experimental import pallas as pl
from jax.experimental.pallas import tpu as pltpu
from jax.experimental.pallas import tpu_sc as plsc

_N = 10000
_E = 320000
_D = 128
_C = 40
_CP = 48
_NC = 2
_NS = 16
_NW = _NC * _NS
_EPT = _E // _NW
_CHUNK = 128
_FULL = _EPT // _CHUNK
_REM = _EPT - _FULL * _CHUNK
_RPS = _N // _NS
_BR = 2000


def _make_sc_agg(width, do_gather):
  mesh = plsc.VectorSubcoreMesh(
      core_axis_name="c", subcore_axis_name="s",
      num_cores=_NC, num_subcores=_NS)
  nlane = width // 16

  scratch = [
      pltpu.VMEM((2, _CHUNK), jnp.int32),
      pltpu.VMEM((2, _REM), jnp.int32),
      pltpu.VMEM((_CHUNK, width), jnp.float32),
      pltpu.VMEM((_REM, width), jnp.float32),
      pltpu.VMEM_SHARED((_N, width), jnp.float32),
  ]

  def body(*refs):
    if do_gather:
      (g_hbm, ei_hbm, out_hbm, idx_v, idxr_v, rows_v, rowsr_v, acc) = refs
    else:
      (ei_hbm, out_hbm, idx_v, idxr_v, rows_v, rowsr_v, acc) = refs
    c = lax.axis_index("c")
    s = lax.axis_index("s")
    tile = c * _NS + s

    zero16 = jnp.zeros((16,), jnp.float32)

    def fill_zero(i, _):
      for j in range(nlane):
        rows_v[i, pl.ds(j * 16, 16)] = zero16
      return 0
    lax.fori_loop(0, _CHUNK, fill_zero, 0)

    r = 0
    while r < _RPS:
      nr = min(_CHUNK, _RPS - r)
      pltpu.sync_copy(rows_v.at[pl.ds(0, nr)], acc.at[pl.ds(s * _RPS + r, nr)])
      r += nr

    if not do_gather:
      e0 = jnp.where(lax.iota(jnp.int32, 16) == 0, 1.0, 0.0)

      def fill_e0(i, _):
        rows_v[i, pl.ds(0, 16)] = e0
        for j in range(1, nlane):
          rows_v[i, pl.ds(j * 16, 16)] = zero16
        return 0
      lax.fori_loop(0, _CHUNK, fill_e0, 0)
      for i in range(_REM):
        rowsr_v[i, pl.ds(0, 16)] = e0
        for j in range(1, nlane):
          rowsr_v[i, pl.ds(j * 16, 16)] = zero16

    plsc.subcore_barrier()

    base = tile * _EPT

    def chunk_body(j, _):
      off = base + j * _CHUNK
      pltpu.sync_copy(ei_hbm.at[:, pl.ds(off, _CHUNK)], idx_v)
      if do_gather:
        pltpu.sync_copy(g_hbm.at[idx_v.at[0]], rows_v)
      pltpu.sync_copy(rows_v, acc.at[idx_v.at[1]], add=True)
      return 0
    lax.fori_loop(0, _FULL, chunk_body, 0)

    offr = base + _FULL * _CHUNK
    pltpu.sync_copy(ei_hbm.at[:, pl.ds(offr, _REM)], idxr_v)
    if do_gather:
      pltpu.sync_copy(g_hbm.at[idxr_v.at[0]], rowsr_v)
    pltpu.sync_copy(rowsr_v, acc.at[idxr_v.at[1]], add=True)

    plsc.subcore_barrier()

    r = 0
    while r < _RPS:
      nr = min(_CHUNK, _RPS - r)
      row0 = s * _RPS + r
      pltpu.sync_copy(acc.at[pl.ds(row0, nr)], rows_v.at[pl.ds(0, nr)])
      pltpu.sync_copy(rows_v.at[pl.ds(0, nr)], out_hbm.at[c, pl.ds(row0, nr)])
      r += nr

  return pl.kernel(
      body,
      out_type=jax.ShapeDtypeStruct((_NC, _N, width), jnp.float32),
      mesh=mesh,
      scratch_types=scratch,
  )


_sc_deg = _make_sc_agg(16, do_gather=False)
_sc_agg128 = _make_sc_agg(_D, do_gather=True)
_sc_agg48 = _make_sc_agg(_CP, do_gather=True)


def _mm1_body(x_ref, w_ref, degp_ref, g_ref, dinv_ref):
  deg = degp_ref[0, :, 0:1] + degp_ref[1, :, 0:1] + 1.0
  dinv = lax.rsqrt(deg)
  dinv_ref[...] = dinv
  g_ref[...] = jnp.dot(x_ref[...], w_ref[...],
                       preferred_element_type=jnp.float32) * dinv


def _mm1(x, W1, degp):
  grid = _N // _BR
  return pl.pallas_call(
      _mm1_body,
      grid=(grid,),
      in_specs=[
          pl.BlockSpec((_BR, _D), lambda i: (i, 0)),
          pl.BlockSpec((_D, _D), lambda i: (0, 0)),
          pl.BlockSpec((_NC, _BR, 16), lambda i: (0, i, 0)),
      ],
      out_specs=[
          pl.BlockSpec((_BR, _D), lambda i: (i, 0)),
          pl.BlockSpec((_BR, 1), lambda i: (i, 0)),
      ],
      out_shape=[
          jax.ShapeDtypeStruct((_N, _D), jnp.float32),
          jax.ShapeDtypeStruct((_N, 1), jnp.float32),
      ],
  )(x, W1, degp)


def _layer_body(p_ref, g_ref, dinv_ref, w_ref, b_ref, out_ref):
  dinv = dinv_ref[...]
  pre = (p_ref[0] + p_ref[1] + g_ref[...]) * dinv + b_ref[...]
  h = jnp.maximum(pre, 0.0)
  out_ref[...] = jnp.dot(h, w_ref[...],
                         preferred_element_type=jnp.float32) * dinv


def _layer(p, g, dinv, W, b, wout):
  grid = _N // _BR
  return pl.pallas_call(
      _layer_body,
      grid=(grid,),
      in_specs=[
          pl.BlockSpec((_NC, _BR, _D), lambda i: (0, i, 0)),
          pl.BlockSpec((_BR, _D), lambda i: (i, 0)),
          pl.BlockSpec((_BR, 1), lambda i: (i, 0)),
          pl.BlockSpec((_D, wout), lambda i: (0, 0)),
          pl.BlockSpec((1, _D), lambda i: (0, 0)),
      ],
      out_specs=pl.BlockSpec((_BR, wout), lambda i: (i, 0)),
      out_shape=jax.ShapeDtypeStruct((_N, wout), jnp.float32),
  )(p, g, dinv, W, b)


def _loss_body(p_ref, g_ref, dinv_ref, b_ref, y_ref, m_ref, loss_ref, acc_ref):
  i = pl.program_id(0)
  l = (p_ref[0] + p_ref[1] + g_ref[...]) * dinv_ref[...] + b_ref[...]
  col = lax.broadcasted_iota(jnp.int32, l.shape, 1)
  lm = jnp.where(col < _C, l, -1e30)
  mx = jnp.max(lm, axis=1, keepdims=True)
  ex = jnp.where(col < _C, jnp.exp(lm - mx), 0.0)
  lse = jnp.log(jnp.sum(ex, axis=1, keepdims=True))
  y = y_ref[...]
  ly = jnp.sum(jnp.where(col == y, l, 0.0), axis=1, keepdims=True)
  valid = m_ref[...] * jnp.where(y != -1, 1.0, 0.0)
  per = mx + lse - ly
  sn = jnp.sum(per * valid)
  sd = jnp.sum(valid)

  @pl.when(i == 0)
  def _():
    acc_ref[0] = 0.0
    acc_ref[1] = 0.0

  acc_ref[0] += sn
  acc_ref[1] += sd

  @pl.when(i == pl.num_programs(0) - 1)
  def _():
    loss_ref[0, 0] = acc_ref[0] / jnp.maximum(acc_ref[1], 1.0)


def _loss(p, g, dinv, b, y, m):
  grid = _N // _BR
  return pl.pallas_call(
      _loss_body,
      grid=(grid,),
      in_specs=[
          pl.BlockSpec((_NC, _BR, _CP), lambda i: (0, i, 0)),
          pl.BlockSpec((_BR, _CP), lambda i: (i, 0)),
          pl.BlockSpec((_BR, 1), lambda i: (i, 0)),
          pl.BlockSpec((1, _CP), lambda i: (0, 0)),
          pl.BlockSpec((_BR, 1), lambda i: (i, 0)),
          pl.BlockSpec((_BR, 1), lambda i: (i, 0)),
      ],
      out_specs=pl.BlockSpec((1, 1), lambda i: (0, 0)),
      out_shape=jax.ShapeDtypeStruct((1, 1), jnp.float32),
      scratch_shapes=[pltpu.SMEM((2,), jnp.float32)],
  )(p, g, dinv, b, y, m)


def kernel(x, edge_index, y, train_mask, W1, b1, W2, b2, W3, b3):
  ei = edge_index.astype(jnp.int32)
  y32 = y.astype(jnp.int32).reshape(_N, 1)
  mf = train_mask.astype(jnp.float32).reshape(_N, 1)
  b1r = b1.reshape(1, _D)
  b2r = b2.reshape(1, _D)
  W3p = jnp.pad(W3, ((0, 0), (0, _CP - _C)))
  b3p = jnp.pad(b3, (0, _CP - _C)).reshape(1, _CP)

  degp = _sc_deg(ei)
  g1, dinv = _mm1(x, W1, degp)
  p1 = _sc_agg128(g1, ei)
  g2 = _layer(p1, g1, dinv, W2, b1r, _D)
  p2 = _sc_agg128(g2, ei)
  g3 = _layer(p2, g2, dinv, W3p, b2r, _CP)
  p3 = _sc_agg48(g3, ei)
  loss = _loss(p3, g3, dinv, b3p, y32, mf)
  return loss.reshape(())


# speedup vs baseline: 15.2462x; 15.2462x over previous
import functools

import jax
import jax.numpy as jnp
from jax import lax
from jax.experimental import pallas as pl
from jax.experimental.pallas import tpu as pltpu
from jax.experimental.pallas import tpu_sc as plsc

_N = 10000
_E = 320000
_D = 128
_C = 40
_CP = 128
_NC = 2
_NS = 16
_NW = _NC * _NS
_CHUNK = 128
_NCHUNK = _E // _CHUNK
_CPT = _NCHUNK // _NW
_XTRA = _NCHUNK - _CPT * _NW
_RPS = 632
_RPS_LAST = _N - 15 * _RPS
_BR = 2000


@functools.lru_cache(maxsize=None)
def _make_sc_agg(width, do_gather):
  mesh = plsc.VectorSubcoreMesh(
      core_axis_name="c", subcore_axis_name="s",
      num_cores=_NC, num_subcores=_NS)
  nlane = width // 16

  scratch = [
      pltpu.VMEM((2, _CHUNK), jnp.int32),
      pltpu.VMEM((_CHUNK, width), jnp.float32),
      pltpu.VMEM_SHARED((_N, width), jnp.float32),
  ]

  _blocks = [(0, 128), (128, 128), (256, 128), (384, 128)]
  _blocks_main = _blocks + [(512, _RPS - 512)]
  _blocks_last = _blocks + [(512, _RPS_LAST - 512)]

  def body(*refs):
    if do_gather:
      (g_hbm, ei_hbm, out_hbm, idx_v, rows_v, acc) = refs
    else:
      (ei_hbm, out_hbm, idx_v, rows_v, acc) = refs
    c = lax.axis_index("c")
    s = lax.axis_index("s")
    tile = c * _NS + s

    zero16 = jnp.zeros((16,), jnp.float32)

    def fill_zero(i, _):
      for j in range(nlane):
        rows_v[i, pl.ds(j * 16, 16)] = zero16
      return 0
    lax.fori_loop(0, _CHUNK, fill_zero, 0)

    def acc_copy(out):
      def run(blocks):
        for r0, nr in blocks:
          row0 = pl.multiple_of(s * _RPS + r0, 8)
          if out == "zero":
            pltpu.sync_copy(rows_v.at[pl.ds(0, nr)], acc.at[pl.ds(row0, nr)])
          else:
            pltpu.sync_copy(acc.at[pl.ds(row0, nr)], rows_v.at[pl.ds(0, nr)])
            pltpu.sync_copy(rows_v.at[pl.ds(0, nr)],
                            out_hbm.at[c, pl.ds(row0, nr)])

      @pl.when(s < _NS - 1)
      def _():
        run(_blocks_main)

      @pl.when(s == _NS - 1)
      def _():
        run(_blocks_last)

    acc_copy("zero")

    if not do_gather:
      e0 = jnp.where(lax.iota(jnp.int32, 16) == 0, 1.0, 0.0)

      def fill_e0(i, _):
        rows_v[i, pl.ds(0, 16)] = e0
        for j in range(1, nlane):
          rows_v[i, pl.ds(j * 16, 16)] = zero16
        return 0
      lax.fori_loop(0, _CHUNK, fill_e0, 0)

    plsc.subcore_barrier()

    nchunks = _CPT + jnp.where(tile < _XTRA, 1, 0)

    def chunk_body(j, _):
      k = j * _NW + tile
      off = pl.multiple_of(k * _CHUNK, _CHUNK)
      pltpu.sync_copy(ei_hbm.at[:, pl.ds(off, _CHUNK)], idx_v)
      if do_gather:
        pltpu.sync_copy(g_hbm.at[idx_v.at[0]], rows_v)
      pltpu.sync_copy(rows_v, acc.at[idx_v.at[1]], add=True)
      return 0
    lax.fori_loop(0, nchunks, chunk_body, 0)

    plsc.subcore_barrier()

    acc_copy("out")

  return pl.kernel(
      body,
      out_type=jax.ShapeDtypeStruct((_NC, _N, width), jnp.float32),
      mesh=mesh,
      scratch_types=scratch,
  )


def _mm1_body(x_ref, w_ref, degp_ref, g_ref, dinv_ref):
  deg = degp_ref[0, :, 0:1] + degp_ref[1, :, 0:1] + 1.0
  dinv = lax.rsqrt(deg)
  dinv_ref[...] = dinv
  g_ref[...] = jnp.dot(x_ref[...], w_ref[...],
                       preferred_element_type=jnp.float32) * dinv


def _mm1(x, W1, degp):
  grid = _N // _BR
  return pl.pallas_call(
      _mm1_body,
      grid=(grid,),
      in_specs=[
          pl.BlockSpec((_BR, _D), lambda i: (i, 0)),
          pl.BlockSpec((_D, _D), lambda i: (0, 0)),
          pl.BlockSpec((_NC, _BR, _D), lambda i: (0, i, 0)),
      ],
      out_specs=[
          pl.BlockSpec((_BR, _D), lambda i: (i, 0)),
          pl.BlockSpec((_BR, 1), lambda i: (i, 0)),
      ],
      out_shape=[
          jax.ShapeDtypeStruct((_N, _D), jnp.float32),
          jax.ShapeDtypeStruct((_N, 1), jnp.float32),
      ],
  )(x, W1, degp)


def _layer_body(p_ref, g_ref, dinv_ref, w_ref, b_ref, out_ref):
  dinv = dinv_ref[...]
  pre = (p_ref[0] + p_ref[1] + g_ref[...]) * dinv + b_ref[...]
  h = jnp.maximum(pre, 0.0)
  out_ref[...] = jnp.dot(h, w_ref[...],
                         preferred_element_type=jnp.float32) * dinv


def _layer(p, g, dinv, W, b, wout):
  grid = _N // _BR
  return pl.pallas_call(
      _layer_body,
      grid=(grid,),
      in_specs=[
          pl.BlockSpec((_NC, _BR, _D), lambda i: (0, i, 0)),
          pl.BlockSpec((_BR, _D), lambda i: (i, 0)),
          pl.BlockSpec((_BR, 1), lambda i: (i, 0)),
          pl.BlockSpec((_D, wout), lambda i: (0, 0)),
          pl.BlockSpec((1, _D), lambda i: (0, 0)),
      ],
      out_specs=pl.BlockSpec((_BR, wout), lambda i: (i, 0)),
      out_shape=jax.ShapeDtypeStruct((_N, wout), jnp.float32),
  )(p, g, dinv, W, b)


def _loss_body(p_ref, g_ref, dinv_ref, b_ref, y_ref, m_ref, loss_ref, acc_ref):
  i = pl.program_id(0)
  l = (p_ref[0] + p_ref[1] + g_ref[...]) * dinv_ref[...] + b_ref[...]
  col = lax.broadcasted_iota(jnp.int32, l.shape, 1)
  lm = jnp.where(col < _C, l, -1e30)
  mx = jnp.max(lm, axis=1, keepdims=True)
  ex = jnp.where(col < _C, jnp.exp(lm - mx), 0.0)
  lse = jnp.log(jnp.sum(ex, axis=1, keepdims=True))
  y = y_ref[...]
  ly = jnp.sum(jnp.where(col == y, l, 0.0), axis=1, keepdims=True)
  valid = m_ref[...] * jnp.where(y != -1, 1.0, 0.0)
  per = mx + lse - ly
  sn = jnp.sum(per * valid)
  sd = jnp.sum(valid)

  @pl.when(i == 0)
  def _():
    acc_ref[0] = 0.0
    acc_ref[1] = 0.0

  acc_ref[0] += sn
  acc_ref[1] += sd

  @pl.when(i == pl.num_programs(0) - 1)
  def _():
    loss_ref[...] = jnp.reshape(acc_ref[0] / jnp.maximum(acc_ref[1], 1.0),
                                (1, 1))


def _loss(p, g, dinv, b, y, m):
  grid = _N // _BR
  return pl.pallas_call(
      _loss_body,
      grid=(grid,),
      in_specs=[
          pl.BlockSpec((_NC, _BR, _CP), lambda i: (0, i, 0)),
          pl.BlockSpec((_BR, _CP), lambda i: (i, 0)),
          pl.BlockSpec((_BR, 1), lambda i: (i, 0)),
          pl.BlockSpec((1, _CP), lambda i: (0, 0)),
          pl.BlockSpec((_BR, 1), lambda i: (i, 0)),
          pl.BlockSpec((_BR, 1), lambda i: (i, 0)),
      ],
      out_specs=pl.BlockSpec((1, 1), lambda i: (0, 0)),
      out_shape=jax.ShapeDtypeStruct((1, 1), jnp.float32),
      scratch_shapes=[pltpu.SMEM((2,), jnp.float32)],
  )(p, g, dinv, b, y, m)


def kernel(x, edge_index, y, train_mask, W1, b1, W2, b2, W3, b3):
  ei = edge_index.astype(jnp.int32)
  y32 = y.astype(jnp.int32).reshape(_N, 1)
  mf = train_mask.astype(jnp.float32).reshape(_N, 1)
  b1r = b1.reshape(1, _D)
  b2r = b2.reshape(1, _D)
  W3p = jnp.pad(W3, ((0, 0), (0, _CP - _C)))
  b3p = jnp.pad(b3, (0, _CP - _C)).reshape(1, _CP)

  sc_deg = _make_sc_agg(_D, False)
  sc_agg128 = _make_sc_agg(_D, True)

  degp = sc_deg(ei)
  g1, dinv = _mm1(x, W1, degp)
  p1 = sc_agg128(g1, ei)
  g2 = _layer(p1, g1, dinv, W2, b1r, _D)
  p2 = sc_agg128(g2, ei)
  g3 = _layer(p2, g2, dinv, W3p, b2r, _CP)
  p3 = sc_agg128(g3, ei)
  loss = _loss(p3, g3, dinv, b3p, y32, mf)
  return loss.reshape(())


# speedup vs baseline: 19.5137x; 1.2799x over previous
import functools

import jax
import jax.numpy as jnp
from jax import lax
from jax.experimental import pallas as pl
from jax.experimental.pallas import tpu as pltpu
from jax.experimental.pallas import tpu_sc as plsc

_N = 10000
_E = 320000
_D = 128
_C = 40
_CP = 128
_NC = 2
_NS = 16
_NW = _NC * _NS
_CHUNK = 128
_NCHUNK = _E // _CHUNK
_CPT = _NCHUNK // _NW
_XTRA = _NCHUNK - _CPT * _NW
_RPS = 632
_RPS_LAST = _N - 15 * _RPS
_BR = 2000


@functools.lru_cache(maxsize=None)
def _make_sc_agg(width, do_gather):
  mesh = plsc.VectorSubcoreMesh(
      core_axis_name="c", subcore_axis_name="s",
      num_cores=_NC, num_subcores=_NS)
  nlane = width // 16

  scratch = [
      pltpu.VMEM((2, _CHUNK), jnp.int32),
      pltpu.VMEM((2, _CHUNK), jnp.int32),
      pltpu.VMEM((_CHUNK, width), jnp.float32),
      pltpu.VMEM((_CHUNK, width), jnp.float32),
      pltpu.VMEM_SHARED((_N, width), jnp.float32),
      pltpu.SemaphoreType.DMA,
      pltpu.SemaphoreType.DMA,
      pltpu.SemaphoreType.DMA,
      pltpu.SemaphoreType.DMA,
  ]

  _blocks = [(0, 128), (128, 128), (256, 128), (384, 128)]
  _blocks_main = _blocks + [(512, _RPS - 512)]
  _blocks_last = _blocks + [(512, _RPS_LAST - 512)]

  def body(*refs):
    if do_gather:
      (g_hbm, ei_hbm, out_hbm, idx0_v, idx1_v, rows0_v, rows1_v, acc,
       semg0, semg1, sems0, sems1) = refs
    else:
      (ei_hbm, out_hbm, idx0_v, idx1_v, rows0_v, rows1_v, acc,
       semg0, semg1, sems0, sems1) = refs
    idx_b = (idx0_v, idx1_v)
    rows_b = (rows0_v, rows1_v)
    semg_b = (semg0, semg1)
    sems_b = (sems0, sems1)
    idx_v, rows_v = idx0_v, rows0_v
    c = lax.axis_index("c")
    s = lax.axis_index("s")
    tile = c * _NS + s

    zero16 = jnp.zeros((16,), jnp.float32)

    def fill_zero(i, _):
      for j in range(nlane):
        rows_v[i, pl.ds(j * 16, 16)] = zero16
      return 0
    lax.fori_loop(0, _CHUNK, fill_zero, 0)

    def acc_copy(out):
      def run(blocks):
        for r0, nr in blocks:
          row0 = pl.multiple_of(s * _RPS + r0, 8)
          if out == "zero":
            pltpu.sync_copy(rows_v.at[pl.ds(0, nr)], acc.at[pl.ds(row0, nr)])
          else:
            pltpu.sync_copy(acc.at[pl.ds(row0, nr)], rows_v.at[pl.ds(0, nr)])
            pltpu.sync_copy(rows_v.at[pl.ds(0, nr)],
                            out_hbm.at[c, pl.ds(row0, nr)])

      @pl.when(s < _NS - 1)
      def _():
        run(_blocks_main)

      @pl.when(s == _NS - 1)
      def _():
        run(_blocks_last)

    acc_copy("zero")

    if not do_gather:
      e0 = jnp.where(lax.iota(jnp.int32, 16) == 0, 1.0, 0.0)

      def fill_e0(i, _):
        rows_v[i, pl.ds(0, 16)] = e0
        for j in range(1, nlane):
          rows_v[i, pl.ds(j * 16, 16)] = zero16
        return 0
      lax.fori_loop(0, _CHUNK, fill_e0, 0)

    plsc.subcore_barrier()

    def load_idx(b, j):
      off = pl.multiple_of((j * _NW + tile) * _CHUNK, _CHUNK)
      pltpu.sync_copy(ei_hbm.at[:, pl.ds(off, _CHUNK)], idx_b[b])

    def gat(b):
      return pltpu.make_async_copy(g_hbm.at[idx_b[b].at[0]], rows_b[b],
                                   semg_b[b])

    def scat(b):
      src = rows_b[b] if do_gather else rows0_v
      return pltpu.make_async_copy(src, acc.at[idx_b[b].at[1]], sems_b[b])

    npair = _CPT // 2

    load_idx(0, 0)
    if do_gather:
      gat(0).start()

      def pipe(i, _):
        for b, jj in ((0, 2 * i), (1, 2 * i + 1)):
          gat(b).wait()
          if b == 0:
            @pl.when(i > 0)
            def _():
              scat(1).wait()
          else:
            scat(0).wait()
          scat(b).start(add=True)
          if b == 0:
            load_idx(1, jj + 1)
            gat(1).start()
          else:
            @pl.when(i < npair - 1)
            def _():
              load_idx(0, jj + 1)
              gat(0).start()
        return 0
    else:

      def pipe(i, _):
        for b, jj in ((0, 2 * i), (1, 2 * i + 1)):
          scat(b).start(add=True)
          if b == 0:
            @pl.when(i > 0)
            def _():
              scat(1).wait()
            load_idx(1, jj + 1)
          else:
            scat(0).wait()

            @pl.when(i < npair - 1)
            def _():
              load_idx(0, jj + 1)
        return 0

    lax.fori_loop(0, npair, pipe, 0)
    scat(1).wait()

    @pl.when(tile < _XTRA)
    def _():
      off = pl.multiple_of((_CPT * _NW + tile) * _CHUNK, _CHUNK)
      pltpu.sync_copy(ei_hbm.at[:, pl.ds(off, _CHUNK)], idx0_v)
      if do_gather:
        pltpu.sync_copy(g_hbm.at[idx0_v.at[0]], rows0_v)
      pltpu.sync_copy(rows0_v, acc.at[idx0_v.at[1]], add=True)

    plsc.subcore_barrier()

    acc_copy("out")

  return pl.kernel(
      body,
      out_type=jax.ShapeDtypeStruct((_NC, _N, width), jnp.float32),
      mesh=mesh,
      scratch_types=scratch,
  )


def _mm1_body(x_ref, w_ref, degp_ref, g_ref, dinv_ref):
  deg = degp_ref[0, :, 0:1] + degp_ref[1, :, 0:1] + 1.0
  dinv = lax.rsqrt(deg)
  dinv_ref[...] = dinv
  g_ref[...] = jnp.dot(x_ref[...], w_ref[...],
                       preferred_element_type=jnp.float32) * dinv


def _mm1(x, W1, degp):
  grid = _N // _BR
  return pl.pallas_call(
      _mm1_body,
      grid=(grid,),
      in_specs=[
          pl.BlockSpec((_BR, _D), lambda i: (i, 0)),
          pl.BlockSpec((_D, _D), lambda i: (0, 0)),
          pl.BlockSpec((_NC, _BR, _D), lambda i: (0, i, 0)),
      ],
      out_specs=[
          pl.BlockSpec((_BR, _D), lambda i: (i, 0)),
          pl.BlockSpec((_BR, 1), lambda i: (i, 0)),
      ],
      out_shape=[
          jax.ShapeDtypeStruct((_N, _D), jnp.float32),
          jax.ShapeDtypeStruct((_N, 1), jnp.float32),
      ],
  )(x, W1, degp)


def _layer_body(p_ref, g_ref, dinv_ref, w_ref, b_ref, out_ref):
  dinv = dinv_ref[...]
  pre = (p_ref[0] + p_ref[1] + g_ref[...]) * dinv + b_ref[...]
  h = jnp.maximum(pre, 0.0)
  out_ref[...] = jnp.dot(h, w_ref[...],
                         preferred_element_type=jnp.float32) * dinv


def _layer(p, g, dinv, W, b, wout):
  grid = _N // _BR
  return pl.pallas_call(
      _layer_body,
      grid=(grid,),
      in_specs=[
          pl.BlockSpec((_NC, _BR, _D), lambda i: (0, i, 0)),
          pl.BlockSpec((_BR, _D), lambda i: (i, 0)),
          pl.BlockSpec((_BR, 1), lambda i: (i, 0)),
          pl.BlockSpec((_D, wout), lambda i: (0, 0)),
          pl.BlockSpec((1, _D), lambda i: (0, 0)),
      ],
      out_specs=pl.BlockSpec((_BR, wout), lambda i: (i, 0)),
      out_shape=jax.ShapeDtypeStruct((_N, wout), jnp.float32),
  )(p, g, dinv, W, b)


def _loss_body(p_ref, g_ref, dinv_ref, b_ref, y_ref, m_ref, loss_ref, acc_ref):
  i = pl.program_id(0)
  l = (p_ref[0] + p_ref[1] + g_ref[...]) * dinv_ref[...] + b_ref[...]
  col = lax.broadcasted_iota(jnp.int32, l.shape, 1)
  lm = jnp.where(col < _C, l, -1e30)
  mx = jnp.max(lm, axis=1, keepdims=True)
  ex = jnp.where(col < _C, jnp.exp(lm - mx), 0.0)
  lse = jnp.log(jnp.sum(ex, axis=1, keepdims=True))
  y = y_ref[...]
  ly = jnp.sum(jnp.where(col == y, l, 0.0), axis=1, keepdims=True)
  valid = m_ref[...] * jnp.where(y != -1, 1.0, 0.0)
  per = mx + lse - ly
  sn = jnp.sum(per * valid)
  sd = jnp.sum(valid)

  @pl.when(i == 0)
  def _():
    acc_ref[0] = 0.0
    acc_ref[1] = 0.0

  acc_ref[0] += sn
  acc_ref[1] += sd

  @pl.when(i == pl.num_programs(0) - 1)
  def _():
    loss_ref[...] = jnp.reshape(acc_ref[0] / jnp.maximum(acc_ref[1], 1.0),
                                (1, 1))


def _loss(p, g, dinv, b, y, m):
  grid = _N // _BR
  return pl.pallas_call(
      _loss_body,
      grid=(grid,),
      in_specs=[
          pl.BlockSpec((_NC, _BR, _CP), lambda i: (0, i, 0)),
          pl.BlockSpec((_BR, _CP), lambda i: (i, 0)),
          pl.BlockSpec((_BR, 1), lambda i: (i, 0)),
          pl.BlockSpec((1, _CP), lambda i: (0, 0)),
          pl.BlockSpec((_BR, 1), lambda i: (i, 0)),
          pl.BlockSpec((_BR, 1), lambda i: (i, 0)),
      ],
      out_specs=pl.BlockSpec((1, 1), lambda i: (0, 0)),
      out_shape=jax.ShapeDtypeStruct((1, 1), jnp.float32),
      scratch_shapes=[pltpu.SMEM((2,), jnp.float32)],
  )(p, g, dinv, b, y, m)


def kernel(x, edge_index, y, train_mask, W1, b1, W2, b2, W3, b3):
  ei = edge_index.astype(jnp.int32)
  y32 = y.astype(jnp.int32).reshape(_N, 1)
  mf = train_mask.astype(jnp.float32).reshape(_N, 1)
  b1r = b1.reshape(1, _D)
  b2r = b2.reshape(1, _D)
  W3p = jnp.pad(W3, ((0, 0), (0, _CP - _C)))
  b3p = jnp.pad(b3, (0, _CP - _C)).reshape(1, _CP)

  sc_deg = _make_sc_agg(_D, False)
  sc_agg128 = _make_sc_agg(_D, True)

  degp = sc_deg(ei)
  g1, dinv = _mm1(x, W1, degp)
  p1 = sc_agg128(g1, ei)
  g2 = _layer(p1, g1, dinv, W2, b1r, _D)
  p2 = sc_agg128(g2, ei)
  g3 = _layer(p2, g2, dinv, W3p, b2r, _CP)
  p3 = sc_agg128(g3, ei)
  loss = _loss(p3, g3, dinv, b3p, y32, mf)
  return loss.reshape(())


# speedup vs baseline: 23.5951x; 1.2092x over previous
import functools

import jax
import jax.numpy as jnp
from jax import lax
from jax.experimental import pallas as pl
from jax.experimental.pallas import tpu as pltpu
from jax.experimental.pallas import tpu_sc as plsc

_N = 10000
_E = 320000
_D = 128
_C = 40
_CP = 128
_NC = 2
_NS = 16
_NW = _NC * _NS
_CHUNK = 128
_NCHUNK = _E // _CHUNK
_CPT = _NCHUNK // _NW
_XTRA = _NCHUNK - _CPT * _NW
_RPS = 632
_RPS_LAST = _N - 15 * _RPS
_BR = 2000


@functools.lru_cache(maxsize=None)
def _make_sc_agg(width, do_gather):
  mesh = plsc.VectorSubcoreMesh(
      core_axis_name="c", subcore_axis_name="s",
      num_cores=_NC, num_subcores=_NS)
  nlane = width // 16

  scratch = (
      [pltpu.VMEM((2, _CHUNK), jnp.int32) for _ in range(4)]
      + [pltpu.VMEM((_CHUNK, width), jnp.float32) for _ in range(2)]
      + [pltpu.VMEM_SHARED((_N, width), jnp.float32)]
      + [pltpu.SemaphoreType.DMA for _ in range(8)]
  )

  _blocks = [(0, 128), (128, 128), (256, 128), (384, 128)]
  _blocks_main = _blocks + [(512, _RPS - 512)]
  _blocks_last = _blocks + [(512, _RPS_LAST - 512)]

  def body(*refs):
    if do_gather:
      g_hbm, ei_hbm, out_hbm = refs[:3]
      rest = refs[3:]
    else:
      ei_hbm, out_hbm = refs[:2]
      rest = refs[2:]
    idx_b = rest[0:4]
    rows_b = rest[4:6]
    acc = rest[6]
    semi_b = rest[7:11]
    semg_b = rest[11:13]
    sems_b = rest[13:15]
    rows_v = rows_b[0]
    c = lax.axis_index("c")
    s = lax.axis_index("s")
    tile = c * _NS + s

    zero16 = jnp.zeros((16,), jnp.float32)

    def fill_zero(i, _):
      for j in range(nlane):
        rows_v[i, pl.ds(j * 16, 16)] = zero16
      return 0
    lax.fori_loop(0, _CHUNK, fill_zero, 0)

    def acc_copy(out):
      def run(blocks):
        for r0, nr in blocks:
          row0 = pl.multiple_of(s * _RPS + r0, 8)
          if out == "zero":
            pltpu.sync_copy(rows_v.at[pl.ds(0, nr)], acc.at[pl.ds(row0, nr)])
          else:
            pltpu.sync_copy(acc.at[pl.ds(row0, nr)], rows_v.at[pl.ds(0, nr)])
            pltpu.sync_copy(rows_v.at[pl.ds(0, nr)],
                            out_hbm.at[c, pl.ds(row0, nr)])

      @pl.when(s < _NS - 1)
      def _():
        run(_blocks_main)

      @pl.when(s == _NS - 1)
      def _():
        run(_blocks_last)

    acc_copy("zero")

    if not do_gather:
      e0 = jnp.where(lax.iota(jnp.int32, 16) == 0, 1.0, 0.0)

      def fill_e0(i, _):
        rows_v[i, pl.ds(0, 16)] = e0
        for j in range(1, nlane):
          rows_v[i, pl.ds(j * 16, 16)] = zero16
        return 0
      lax.fori_loop(0, _CHUNK, fill_e0, 0)

    def idxl(j, q):
      off = pl.multiple_of((j * _NW + tile) * _CHUNK, _CHUNK)
      return pltpu.make_async_copy(ei_hbm.at[:, pl.ds(off, _CHUNK)],
                                   idx_b[q], semi_b[q])

    def gat(q, r):
      return pltpu.make_async_copy(g_hbm.at[idx_b[q].at[0]], rows_b[r],
                                   semg_b[r])

    def scat(q, r):
      src = rows_b[r] if do_gather else rows_b[0]
      return pltpu.make_async_copy(src, acc.at[idx_b[q].at[1]], sems_b[r])

    def step(j, t, i=None, tail=False):
      q, qn, qn2 = t % 4, (t + 1) % 4, (t + 2) % 4
      r, rn = t % 2, (t + 1) % 2
      qp, rp = (t - 1) % 4, (t + 1) % 2
      if do_gather:
        gat(q, r).wait()
      if t == 0 and i is not None:
        @pl.when(i > 0)
        def _():
          scat(qp, rp).wait()
      else:
        scat(qp, rp).wait()
      scat(q, r).start(add=True)
      if not (tail and t == 1):
        idxl(j + 1, qn).wait()
        if do_gather:
          gat(qn, rn).start()
      if not tail:
        idxl(j + 2, qn2).start()

    idxl(0, 0).start()
    idxl(1, 1).start()
    idxl(0, 0).wait()
    if do_gather:
      gat(0, 0).start()

    plsc.subcore_barrier()

    def pipe(i, _):
      for t in range(4):
        step(4 * i + t, t, i=i)
      return 0
    lax.fori_loop(0, (_CPT - 2) // 4, pipe, 0)
    step(_CPT - 2, 0, tail=True)
    step(_CPT - 1, 1, tail=True)
    scat(1, 1).wait()

    @pl.when(tile < _XTRA)
    def _():
      off = pl.multiple_of((_CPT * _NW + tile) * _CHUNK, _CHUNK)
      pltpu.sync_copy(ei_hbm.at[:, pl.ds(off, _CHUNK)], idx_b[0])
      if do_gather:
        pltpu.sync_copy(g_hbm.at[idx_b[0].at[0]], rows_b[0])
      pltpu.sync_copy(rows_b[0], acc.at[idx_b[0].at[1]], add=True)

    plsc.subcore_barrier()

    acc_copy("out")

  return pl.kernel(
      body,
      out_type=jax.ShapeDtypeStruct((_NC, _N, width), jnp.float32),
      mesh=mesh,
      scratch_types=scratch,
  )


def _mm1_body(x_ref, w_ref, degp_ref, g_ref, dinv_ref):
  deg = degp_ref[0, :, 0:1] + degp_ref[1, :, 0:1] + 1.0
  dinv = lax.rsqrt(deg)
  dinv_ref[...] = dinv
  g_ref[...] = jnp.dot(x_ref[...], w_ref[...],
                       preferred_element_type=jnp.float32) * dinv


def _mm1(x, W1, degp):
  grid = _N // _BR
  return pl.pallas_call(
      _mm1_body,
      grid=(grid,),
      in_specs=[
          pl.BlockSpec((_BR, _D), lambda i: (i, 0)),
          pl.BlockSpec((_D, _D), lambda i: (0, 0)),
          pl.BlockSpec((_NC, _BR, _D), lambda i: (0, i, 0)),
      ],
      out_specs=[
          pl.BlockSpec((_BR, _D), lambda i: (i, 0)),
          pl.BlockSpec((_BR, 1), lambda i: (i, 0)),
      ],
      out_shape=[
          jax.ShapeDtypeStruct((_N, _D), jnp.float32),
          jax.ShapeDtypeStruct((_N, 1), jnp.float32),
      ],
  )(x, W1, degp)


def _layer_body(p_ref, g_ref, dinv_ref, w_ref, b_ref, out_ref):
  dinv = dinv_ref[...]
  pre = (p_ref[0] + p_ref[1] + g_ref[...]) * dinv + b_ref[...]
  h = jnp.maximum(pre, 0.0)
  out_ref[...] = jnp.dot(h, w_ref[...],
                         preferred_element_type=jnp.float32) * dinv


def _layer(p, g, dinv, W, b, wout):
  grid = _N // _BR
  return pl.pallas_call(
      _layer_body,
      grid=(grid,),
      in_specs=[
          pl.BlockSpec((_NC, _BR, _D), lambda i: (0, i, 0)),
          pl.BlockSpec((_BR, _D), lambda i: (i, 0)),
          pl.BlockSpec((_BR, 1), lambda i: (i, 0)),
          pl.BlockSpec((_D, wout), lambda i: (0, 0)),
          pl.BlockSpec((1, _D), lambda i: (0, 0)),
      ],
      out_specs=pl.BlockSpec((_BR, wout), lambda i: (i, 0)),
      out_shape=jax.ShapeDtypeStruct((_N, wout), jnp.float32),
  )(p, g, dinv, W, b)


def _loss_body(p_ref, g_ref, dinv_ref, b_ref, y_ref, m_ref, loss_ref, acc_ref):
  i = pl.program_id(0)
  l = (p_ref[0] + p_ref[1] + g_ref[...]) * dinv_ref[...] + b_ref[...]
  col = lax.broadcasted_iota(jnp.int32, l.shape, 1)
  lm = jnp.where(col < _C, l, -1e30)
  mx = jnp.max(lm, axis=1, keepdims=True)
  ex = jnp.where(col < _C, jnp.exp(lm - mx), 0.0)
  lse = jnp.log(jnp.sum(ex, axis=1, keepdims=True))
  y = y_ref[...]
  ly = jnp.sum(jnp.where(col == y, l, 0.0), axis=1, keepdims=True)
  valid = m_ref[...] * jnp.where(y != -1, 1.0, 0.0)
  per = mx + lse - ly
  sn = jnp.sum(per * valid)
  sd = jnp.sum(valid)

  @pl.when(i == 0)
  def _():
    acc_ref[0] = 0.0
    acc_ref[1] = 0.0

  acc_ref[0] += sn
  acc_ref[1] += sd

  @pl.when(i == pl.num_programs(0) - 1)
  def _():
    loss_ref[...] = jnp.reshape(acc_ref[0] / jnp.maximum(acc_ref[1], 1.0),
                                (1, 1))


def _loss(p, g, dinv, b, y, m):
  grid = _N // _BR
  return pl.pallas_call(
      _loss_body,
      grid=(grid,),
      in_specs=[
          pl.BlockSpec((_NC, _BR, _CP), lambda i: (0, i, 0)),
          pl.BlockSpec((_BR, _CP), lambda i: (i, 0)),
          pl.BlockSpec((_BR, 1), lambda i: (i, 0)),
          pl.BlockSpec((1, _CP), lambda i: (0, 0)),
          pl.BlockSpec((_BR, 1), lambda i: (i, 0)),
          pl.BlockSpec((_BR, 1), lambda i: (i, 0)),
      ],
      out_specs=pl.BlockSpec((1, 1), lambda i: (0, 0)),
      out_shape=jax.ShapeDtypeStruct((1, 1), jnp.float32),
      scratch_shapes=[pltpu.SMEM((2,), jnp.float32)],
  )(p, g, dinv, b, y, m)


def kernel(x, edge_index, y, train_mask, W1, b1, W2, b2, W3, b3):
  ei = edge_index.astype(jnp.int32)
  y32 = y.astype(jnp.int32).reshape(_N, 1)
  mf = train_mask.astype(jnp.float32).reshape(_N, 1)
  b1r = b1.reshape(1, _D)
  b2r = b2.reshape(1, _D)
  W3p = jnp.pad(W3, ((0, 0), (0, _CP - _C)))
  b3p = jnp.pad(b3, (0, _CP - _C)).reshape(1, _CP)

  sc_deg = _make_sc_agg(_D, False)
  sc_agg128 = _make_sc_agg(_D, True)

  degp = sc_deg(ei)
  g1, dinv = _mm1(x, W1, degp)
  p1 = sc_agg128(g1, ei)
  g2 = _layer(p1, g1, dinv, W2, b1r, _D)
  p2 = sc_agg128(g2, ei)
  g3 = _layer(p2, g2, dinv, W3p, b2r, _CP)
  p3 = sc_agg128(g3, ei)
  loss = _loss(p3, g3, dinv, b3p, y32, mf)
  return loss.reshape(())
